# baseline (speedup 1.0000x reference)
import functools

import jax
import jax.numpy as jnp
from jax import lax
from jax.experimental import pallas as pl
from jax.experimental.pallas import tpu as pltpu
from jax.experimental.pallas import tpu_sc as plsc

VOCAB, D, B, C, K = 100000, 128, 16384, 10, 20
NC, NS = 2, 16
NW = NC * NS
EPW = B // NW
EC = 64
NCHUNK = EPW // EC
ROWS_U = EC * C
NG = ROWS_U // 128
NB = EC * K // 128
NBG = EC // 4
NRS = 4
NDV = D // 16


def _hrow(ref, r):
    return [ref[r, pl.ds(d * 16, 16)] for d in range(NDV)]


def _dot_partial(hv, ref, r):
    p = hv[0] * ref[r, pl.ds(0, 16)]
    for d in range(1, NDV):
        p = p + ref[r, pl.ds(d * 16, 16)] * hv[d]
    return p


def _make_sc_fused():
    mesh = plsc.VectorSubcoreMesh(core_axis_name="c", subcore_axis_name="s",
                                  num_cores=NC, num_subcores=NS)

    @functools.partial(
        pl.kernel,
        out_type=[
            jax.ShapeDtypeStruct((B,), jnp.float32),
            jax.ShapeDtypeStruct((B * K,), jnp.float32),
        ],
        mesh=mesh,
        compiler_params=pltpu.CompilerParams(needs_layout_passes=False),
        scratch_types=[
            pltpu.VMEM((EPW * C // 128, 128), jnp.int32),
            pltpu.VMEM((EPW * K // 80, 80), jnp.int32),
            pltpu.VMEM((8, 128), jnp.int32),
            pltpu.VMEM((ROWS_U, D), jnp.float32),
            pltpu.VMEM((EC, D), jnp.float32),
            pltpu.VMEM((128, D), jnp.float32),
            pltpu.VMEM((256,), jnp.float32),
            pltpu.VMEM((EC,), jnp.float32),
            pltpu.VMEM((EC * K,), jnp.float32),
            pltpu.SemaphoreType.DMA,
            pltpu.SemaphoreType.DMA,
            pltpu.SemaphoreType.DMA,
            pltpu.SemaphoreType.DMA,
            pltpu.SemaphoreType.DMA,
            pltpu.SemaphoreType.DMA,
        ],
    )
    def sc_fused(posu2d, posw2d, neg2d, uw, ww, s2_out, ns_out,
                 uidx, negidx, pwidx, rows, hbuf, pwrows, part, s2buf, nsbuf,
                 usem, nsem0, nsem1, nsem2, nsem3, pwsem):
        wid = lax.axis_index("s") * NC + lax.axis_index("c")
        rowidx = jnp.arange(16, dtype=jnp.int32)

        colidx = rowidx * 16

        def flush16(dst_ref, dst_off):
            cols = [plsc.load_gather(part, [colidx + j]) for j in range(16)]
            while len(cols) > 1:
                cols = [a + b for a, b in zip(cols[::2], cols[1::2])]
            dst_ref[pl.ds(dst_off, 16)] = cols[0]

        pltpu.sync_copy(posu2d.at[pl.ds(wid * (EPW * C // 128),
                                        EPW * C // 128)], uidx)
        pltpu.sync_copy(neg2d.at[pl.ds(wid * (EPW * K // 80),
                                       EPW * K // 80)], negidx)
        pltpu.sync_copy(posw2d.at[pl.ds((wid // 2) * 8, 8)], pwidx)

        def neg_fire(i, b, slot):
            sem = (nsem0, nsem1, nsem2, nsem3)[slot]
            return pltpu.async_copy(ww.at[negidx.at[i * NBG + b]],
                                    rows.at[pl.ds(slot * 80, 80)], sem)

        def chunk(i, carry):
            for g in range(3):
                pltpu.async_copy(uw.at[uidx.at[i * NG + g]],
                                 rows.at[pl.ds(g * 128, 128)], usem)
            @pl.when(i % 2 == 0)
            def _():
                pltpu.async_copy(ww.at[pwidx.at[(wid % 2) * (NCHUNK // 2)
                                                + i // 2]], pwrows, pwsem)
            for g in range(NG):
                pltpu.make_async_copy(uw.at[uidx.at[i * NG + g]],
                                      rows.at[pl.ds(g * 128, 128)],
                                      usem).wait()

            def ex(e, c2):
                r0 = e * C
                for d in range(NDV):
                    sl = pl.ds(d * 16, 16)
                    acc = rows[r0, sl]
                    for cc in range(1, C):
                        acc = acc + rows[r0 + cc, sl]
                    hbuf[e, sl] = acc
                return c2
            lax.fori_loop(0, EC, ex, 0)

            @pl.when(i % 2 == 0)
            def _():
                pltpu.make_async_copy(
                    ww.at[pwidx.at[(wid % 2) * (NCHUNK // 2) + i // 2]],
                    pwrows, pwsem).wait()

            for s in range(3):
                neg_fire(i, s, s)

            @pl.when(i < NCHUNK - 1)
            def _():
                for g in range(3, NG):
                    pltpu.async_copy(uw.at[uidx.at[(i + 1) * NG + g]],
                                     rows.at[pl.ds(g * 128, 128)], usem)

            def pgrp(g, c2):
                for q in range(16):
                    le = g * 16 + q
                    hv = _hrow(hbuf, le)
                    part[pl.ds(q * 16, 16)] = _dot_partial(
                        hv, pwrows, (i % 2) * EC + le)
                flush16(s2buf, g * 16)
                return c2
            lax.fori_loop(0, EC // 16, pgrp, 0)
            pltpu.sync_copy(s2buf, s2_out.at[pl.ds(wid * EPW + i * EC, EC)])

            def nwait(b, slot):
                sem = (nsem0, nsem1, nsem2, nsem3)[slot]
                pltpu.make_async_copy(ww.at[negidx.at[i * NBG + b]],
                                      rows.at[pl.ds(slot * 80, 80)],
                                      sem).wait()

            def nbatch(b, c2):
                slotbase = (b % NRS) * 80
                for s in range(NRS):
                    @pl.when(b % NRS == s)
                    def _(s=s):
                        nwait(b, s)

                for q in range(4):
                    hv = _hrow(hbuf, b * 4 + q)
                    for k in range(K):
                        t = q * K + k
                        part[pl.ds((t % 16) * 16, 16)] = _dot_partial(
                            hv, rows, slotbase + q * K + k)
                        if t % 16 == 15:
                            flush16(nsbuf, b * 80 + (t // 16) * 16)

                for s in range(NRS):
                    @pl.when((b % NRS == s) & (b + 3 < NBG))
                    def _(s=s):
                        neg_fire(i, b + 3, (s + 3) % NRS)
                return c2
            lax.fori_loop(0, NBG, nbatch, 0)

            pltpu.sync_copy(nsbuf,
                            ns_out.at[pl.ds(wid * EPW * K + i * EC * K,
                                            EC * K)])
            return carry

        for g in range(3, NG):
            pltpu.async_copy(uw.at[uidx.at[g]],
                             rows.at[pl.ds(g * 128, 128)], usem)
        lax.fori_loop(0, NCHUNK, chunk, 0)

    return sc_fused


def _tc_loss_body(s2_ref, ns_ref, out_ref):
    part = (jnp.sum(jax.nn.log_sigmoid(s2_ref[...]))
            + jnp.sum(jax.nn.log_sigmoid(-ns_ref[...])))
    out_ref[0, 0] = -part


_tc_loss = pl.pallas_call(
    _tc_loss_body,
    out_specs=pl.BlockSpec(memory_space=pltpu.SMEM),
    out_shape=jax.ShapeDtypeStruct((1, 1), jnp.float32),
)


def kernel(pos_u, pos_w, neg_w, u_weight, w_weight):
    posu2d = pos_u.reshape(B * C // 128, 128)
    posw2d = pos_w.reshape(B // 128, 128)
    neg2d = neg_w.reshape(B * K // 80, 80)
    s2, ns = _make_sc_fused()(posu2d, posw2d, neg2d, u_weight, w_weight)
    loss = _tc_loss(s2.reshape(B // 128, 128), ns.reshape(B * K // 128, 128))
    return loss[0, 0]


# speedup vs baseline: 1.0767x; 1.0767x over previous
import functools

import jax
import jax.numpy as jnp
from jax import lax
from jax.experimental import pallas as pl
from jax.experimental.pallas import tpu as pltpu
from jax.experimental.pallas import tpu_sc as plsc

VOCAB, D, B, C, K = 100000, 128, 16384, 10, 20
NC, NS = 2, 16
NW = NC * NS
EPW = B // NW
EC = 64
NCHUNK = EPW // EC
ROWS_U = EC * C
NG = ROWS_U // 128
NB = EC * K // 128
NDV = D // 16


def _hrow(ref, r):
    return [ref[r, pl.ds(d * 16, 16)] for d in range(NDV)]


def _dot_partial(hv, ref, r):
    p0 = hv[0] * ref[r, pl.ds(0, 16)]
    p1 = hv[1] * ref[r, pl.ds(16, 16)]
    for d in range(2, NDV, 2):
        p0 = p0 + ref[r, pl.ds(d * 16, 16)] * hv[d]
        p1 = p1 + ref[r, pl.ds((d + 1) * 16, 16)] * hv[d + 1]
    return p0 + p1


def _make_sc_fused():
    mesh = plsc.VectorSubcoreMesh(core_axis_name="c", subcore_axis_name="s",
                                  num_cores=NC, num_subcores=NS)

    @functools.partial(
        pl.kernel,
        out_type=[
            jax.ShapeDtypeStruct((B,), jnp.float32),
            jax.ShapeDtypeStruct((B * K,), jnp.float32),
        ],
        mesh=mesh,
        compiler_params=pltpu.CompilerParams(needs_layout_passes=False),
        scratch_types=[
            pltpu.VMEM((EPW * C // 128, 128), jnp.int32),
            pltpu.VMEM((EPW * K // 128, 128), jnp.int32),
            pltpu.VMEM((8, 128), jnp.int32),
            pltpu.VMEM((ROWS_U, D), jnp.float32),
            pltpu.VMEM((EC, D), jnp.float32),
            pltpu.VMEM((128, D), jnp.float32),
            pltpu.VMEM((16, 16), jnp.float32),
            pltpu.VMEM((EC,), jnp.float32),
            pltpu.VMEM((EC * K,), jnp.float32),
            pltpu.SemaphoreType.DMA,
            pltpu.SemaphoreType.DMA,
            pltpu.SemaphoreType.DMA,
            pltpu.SemaphoreType.DMA,
            pltpu.SemaphoreType.DMA,
        ],
    )
    def sc_fused(posu2d, posw2d, neg2d, uw, ww, s2_out, ns_out,
                 uidx, negidx, pwidx, rows, hbuf, pwrows, part, s2buf, nsbuf,
                 usem, nsem0, nsem1, nsem2, pwsem):
        wid = lax.axis_index("s") * NC + lax.axis_index("c")
        rowidx = jnp.arange(16, dtype=jnp.int32)

        def flush16(dst_ref, dst_off):
            cols = [plsc.load_gather(part, [rowidx,
                                            jnp.full(16, j, jnp.int32)])
                    for j in range(16)]
            while len(cols) > 1:
                cols = [a + b for a, b in zip(cols[::2], cols[1::2])]
            dst_ref[pl.ds(dst_off, 16)] = cols[0]

        pltpu.sync_copy(posu2d.at[pl.ds(wid * (EPW * C // 128),
                                        EPW * C // 128)], uidx)
        pltpu.sync_copy(neg2d.at[pl.ds(wid * (EPW * K // 128),
                                       EPW * K // 128)], negidx)
        pltpu.sync_copy(posw2d.at[pl.ds((wid // 2) * 8, 8)], pwidx)

        def neg_fire(i, b, slot):
            sem = (nsem0, nsem1, nsem2)[slot]
            return pltpu.async_copy(ww.at[negidx.at[i * NB + b]],
                                    rows.at[pl.ds(slot * 128, 128)], sem)

        def chunk(i, carry):
            for g in range(3):
                pltpu.async_copy(uw.at[uidx.at[i * NG + g]],
                                 rows.at[pl.ds(g * 128, 128)], usem)
            @pl.when(i % 2 == 0)
            def _():
                pltpu.async_copy(ww.at[pwidx.at[(wid % 2) * (NCHUNK // 2)
                                                + i // 2]], pwrows, pwsem)
            for g in range(NG):
                pltpu.make_async_copy(uw.at[uidx.at[i * NG + g]],
                                      rows.at[pl.ds(g * 128, 128)],
                                      usem).wait()

            @pl.when(i % 2 == 0)
            def _():
                pltpu.make_async_copy(
                    ww.at[pwidx.at[(wid % 2) * (NCHUNK // 2) + i // 2]],
                    pwrows, pwsem).wait()

            def ex(e, c2):
                r0 = e * C
                pr = (i % 2) * EC + e
                pacc = None
                for d in range(NDV):
                    sl = pl.ds(d * 16, 16)
                    acc = rows[r0, sl]
                    for cc in range(1, C):
                        acc = acc + rows[r0 + cc, sl]
                    hbuf[e, sl] = acc
                    pp = acc * pwrows[pr, sl]
                    pacc = pp if pacc is None else pacc + pp
                part[e % 16, :] = pacc

                @pl.when(e % 16 == 15)
                def _():
                    flush16(s2buf, e - 15)
                return c2
            lax.fori_loop(0, EC, ex, 0)

            for s in range(3):
                neg_fire(i, s, s)

            @pl.when(i < NCHUNK - 1)
            def _():
                for g in range(3, NG):
                    pltpu.async_copy(uw.at[uidx.at[(i + 1) * NG + g]],
                                     rows.at[pl.ds(g * 128, 128)], usem)

            pltpu.sync_copy(s2buf, s2_out.at[pl.ds(wid * EPW + i * EC, EC)])

            def nwait(b, slot):
                sem = (nsem0, nsem1, nsem2)[slot]
                pltpu.make_async_copy(ww.at[negidx.at[i * NB + b]],
                                      rows.at[pl.ds(slot * 128, 128)],
                                      sem).wait()

            def nbatch(b, c2):
                slotbase = (b % 3) * 128
                for s in range(3):
                    @pl.when(b % 3 == s)
                    def _(s=s):
                        nwait(b, s)

                def ngrp(f, c3):
                    for q in range(16):
                        j = f * 16 + q
                        le = (b * 128 + j) // K
                        hv = _hrow(hbuf, le)
                        part[q, :] = _dot_partial(hv, rows, slotbase + j)
                    flush16(nsbuf, b * 128 + f * 16)
                    return c3
                lax.fori_loop(0, 8, ngrp, 0)

                for s in range(3):
                    @pl.when((b % 3 == s) & (b + 3 < NB))
                    def _(s=s):
                        neg_fire(i, b + 3, s)
                return c2
            lax.fori_loop(0, NB, nbatch, 0)

            pltpu.sync_copy(nsbuf,
                            ns_out.at[pl.ds(wid * EPW * K + i * EC * K,
                                            EC * K)])
            return carry

        for g in range(3, NG):
            pltpu.async_copy(uw.at[uidx.at[g]],
                             rows.at[pl.ds(g * 128, 128)], usem)
        lax.fori_loop(0, NCHUNK, chunk, 0)

    return sc_fused


def _tc_loss_body(s2_ref, ns_ref, out_ref):
    part = (jnp.sum(jax.nn.log_sigmoid(s2_ref[...]))
            + jnp.sum(jax.nn.log_sigmoid(-ns_ref[...])))
    out_ref[0, 0] = -part


_tc_loss = pl.pallas_call(
    _tc_loss_body,
    out_specs=pl.BlockSpec(memory_space=pltpu.SMEM),
    out_shape=jax.ShapeDtypeStruct((1, 1), jnp.float32),
)


def kernel(pos_u, pos_w, neg_w, u_weight, w_weight):
    posu2d = pos_u.reshape(B * C // 128, 128)
    posw2d = pos_w.reshape(B // 128, 128)
    neg2d = neg_w.reshape(B * K // 128, 128)
    s2, ns = _make_sc_fused()(posu2d, posw2d, neg2d, u_weight, w_weight)
    loss = _tc_loss(s2.reshape(B // 128, 128), ns.reshape(B * K // 128, 128))
    return loss[0, 0]


# speedup vs baseline: 1.1534x; 1.0712x over previous
import functools

import jax
import jax.numpy as jnp
from jax import lax
from jax.experimental import pallas as pl
from jax.experimental.pallas import tpu as pltpu
from jax.experimental.pallas import tpu_sc as plsc

VOCAB, D, B, C, K = 100000, 128, 16384, 10, 20
NC, NS = 2, 16
NW = NC * NS
EPW = B // NW
EC = 64
NCHUNK = EPW // EC
ROWS_U = EC * C
NG = ROWS_U // 128
NB = EC * K // 128
NDV = D // 16


def _hrow(ref, r):
    return [ref[r, pl.ds(d * 16, 16)] for d in range(NDV)]


def _dot_partial(hv, ref, r):
    p = hv[0] * ref[r, pl.ds(0, 16)]
    for d in range(1, NDV):
        p = p + ref[r, pl.ds(d * 16, 16)] * hv[d]
    return p


def _make_sc_fused():
    mesh = plsc.VectorSubcoreMesh(core_axis_name="c", subcore_axis_name="s",
                                  num_cores=NC, num_subcores=NS)

    @functools.partial(
        pl.kernel,
        out_type=[
            jax.ShapeDtypeStruct((B,), jnp.float32),
            jax.ShapeDtypeStruct((B * K,), jnp.float32),
        ],
        mesh=mesh,
        compiler_params=pltpu.CompilerParams(needs_layout_passes=False),
        scratch_types=[
            pltpu.VMEM((EPW * C // 128, 128), jnp.int32),
            pltpu.VMEM((EPW * K // 128, 128), jnp.int32),
            pltpu.VMEM((8, 128), jnp.int32),
            pltpu.VMEM((ROWS_U, D), jnp.float32),
            pltpu.VMEM((EC, D), jnp.float32),
            pltpu.VMEM((128, D), jnp.float32),
            pltpu.VMEM((16, 16), jnp.float32),
            pltpu.VMEM((EC,), jnp.float32),
            pltpu.VMEM((EC * K,), jnp.float32),
            pltpu.SemaphoreType.DMA,
            pltpu.SemaphoreType.DMA,
            pltpu.SemaphoreType.DMA,
            pltpu.SemaphoreType.DMA,
            pltpu.SemaphoreType.DMA,
            pltpu.SemaphoreType.DMA,
            pltpu.SemaphoreType.DMA,
            pltpu.SemaphoreType.DMA,
        ],
    )
    def sc_fused(posu2d, posw2d, neg2d, uw, ww, s2_out, ns_out,
                 uidx, negidx, pwidx, rows, hbuf, pwrows, part, s2buf, nsbuf,
                 usem0, usem1, usem2, usemp, nsem0, nsem1, nsem2, pwsem):
        wid = lax.axis_index("s") * NC + lax.axis_index("c")
        rowidx = jnp.arange(16, dtype=jnp.int32)

        def flush16(dst_ref, dst_off):
            cols = [plsc.load_gather(part, [rowidx,
                                            jnp.full(16, j, jnp.int32)])
                    for j in range(16)]
            while len(cols) > 1:
                cols = [a + b for a, b in zip(cols[::2], cols[1::2])]
            dst_ref[pl.ds(dst_off, 16)] = cols[0]

        pltpu.sync_copy(posu2d.at[pl.ds(wid * (EPW * C // 128),
                                        EPW * C // 128)], uidx)
        pltpu.sync_copy(neg2d.at[pl.ds(wid * (EPW * K // 128),
                                       EPW * K // 128)], negidx)
        pltpu.sync_copy(posw2d.at[pl.ds((wid // 2) * 8, 8)], pwidx)

        def neg_fire(i, b, slot):
            sem = (nsem0, nsem1, nsem2)[slot]
            return pltpu.async_copy(ww.at[negidx.at[i * NB + b]],
                                    rows.at[pl.ds(slot * 128, 128)], sem)

        usems = (usem0, usem1, usem2, usemp, usemp)

        def u_wait(i, g):
            pltpu.make_async_copy(uw.at[uidx.at[i * NG + g]],
                                  rows.at[pl.ds(g * 128, 128)],
                                  usems[g]).wait()

        def chunk(i, carry):
            for g in range(3):
                pltpu.async_copy(uw.at[uidx.at[i * NG + g]],
                                 rows.at[pl.ds(g * 128, 128)], usems[g])
            @pl.when(i % 2 == 0)
            def _():
                pltpu.async_copy(ww.at[pwidx.at[(wid % 2) * (NCHUNK // 2)
                                                + i // 2]], pwrows, pwsem)

            def ex(e, c2):
                r0 = e * C
                for d in range(NDV):
                    sl = pl.ds(d * 16, 16)
                    acc = rows[r0, sl]
                    for cc in range(1, C):
                        acc = acc + rows[r0 + cc, sl]
                    hbuf[e, sl] = acc
                return c2
            u_wait(i, 0)
            u_wait(i, 1)
            lax.fori_loop(0, 25, ex, 0)
            u_wait(i, 2)
            lax.fori_loop(25, 38, ex, 0)
            u_wait(i, 3)
            u_wait(i, 4)
            lax.fori_loop(38, EC, ex, 0)

            @pl.when(i % 2 == 0)
            def _():
                pltpu.make_async_copy(
                    ww.at[pwidx.at[(wid % 2) * (NCHUNK // 2) + i // 2]],
                    pwrows, pwsem).wait()

            for s in range(3):
                neg_fire(i, s, s)

            @pl.when(i < NCHUNK - 1)
            def _():
                for g in range(3, NG):
                    pltpu.async_copy(uw.at[uidx.at[(i + 1) * NG + g]],
                                     rows.at[pl.ds(g * 128, 128)], usemp)

            def pgrp(g, c2):
                for q in range(16):
                    le = g * 16 + q
                    hv = _hrow(hbuf, le)
                    part[q, :] = _dot_partial(hv, pwrows,
                                              (i % 2) * EC + le)
                flush16(s2buf, g * 16)
                return c2
            lax.fori_loop(0, EC // 16, pgrp, 0)
            pltpu.sync_copy(s2buf, s2_out.at[pl.ds(wid * EPW + i * EC, EC)])

            def nwait(b, slot):
                sem = (nsem0, nsem1, nsem2)[slot]
                pltpu.make_async_copy(ww.at[negidx.at[i * NB + b]],
                                      rows.at[pl.ds(slot * 128, 128)],
                                      sem).wait()

            def nbatch(b, c2):
                slotbase = (b % 3) * 128
                for s in range(3):
                    @pl.when(b % 3 == s)
                    def _(s=s):
                        nwait(b, s)

                def ngrp(f, c3):
                    for q in range(16):
                        j = f * 16 + q
                        le = (b * 128 + j) // K
                        hv = _hrow(hbuf, le)
                        part[q, :] = _dot_partial(hv, rows, slotbase + j)
                    flush16(nsbuf, b * 128 + f * 16)
                    return c3
                lax.fori_loop(0, 8, ngrp, 0)

                for s in range(3):
                    @pl.when((b % 3 == s) & (b + 3 < NB))
                    def _(s=s):
                        neg_fire(i, b + 3, s)
                return c2
            lax.fori_loop(0, NB, nbatch, 0)

            pltpu.sync_copy(nsbuf,
                            ns_out.at[pl.ds(wid * EPW * K + i * EC * K,
                                            EC * K)])
            return carry

        for g in range(3, NG):
            pltpu.async_copy(uw.at[uidx.at[g]],
                             rows.at[pl.ds(g * 128, 128)], usemp)
        lax.fori_loop(0, NCHUNK, chunk, 0)

    return sc_fused


def _tc_loss_body(s2_ref, ns_ref, out_ref):
    part = (jnp.sum(jax.nn.log_sigmoid(s2_ref[...]))
            + jnp.sum(jax.nn.log_sigmoid(-ns_ref[...])))
    out_ref[0, 0] = -part


_tc_loss = pl.pallas_call(
    _tc_loss_body,
    out_specs=pl.BlockSpec(memory_space=pltpu.SMEM),
    out_shape=jax.ShapeDtypeStruct((1, 1), jnp.float32),
)


def kernel(pos_u, pos_w, neg_w, u_weight, w_weight):
    posu2d = pos_u.reshape(B * C // 128, 128)
    posw2d = pos_w.reshape(B // 128, 128)
    neg2d = neg_w.reshape(B * K // 128, 128)
    s2, ns = _make_sc_fused()(posu2d, posw2d, neg2d, u_weight, w_weight)
    loss = _tc_loss(s2.reshape(B // 128, 128), ns.reshape(B * K // 128, 128))
    return loss[0, 0]
